# Initial kernel scaffold; baseline (speedup 1.0000x reference)
import jax
import jax.numpy as jnp
from jax.experimental import pallas as pl


def kernel(p, params):
    raise NotImplementedError("write your pallas kernel here")


# speedup vs baseline: 204.1888x; 204.1888x over previous
import functools

import jax
import jax.numpy as jnp
from jax import lax
from jax.experimental import pallas as pl
from jax.experimental.pallas import tpu as pltpu
from jax.experimental.pallas import tpu_sc as plsc

BS = 4
NPTS = 100000
MID = 32
RESO = 32
D3 = RESO ** 3
NC = 2
NS = 16
CH = 49
TPB = CH * 128
NP = NS * TPB
TROWS = D3 + 16
RPT = D3 // NS
DEN = 1.0 + 0.1 + 0.001
HI = 1.0 - 0.001
RB = 1024


def _mesh():
    return plsc.VectorSubcoreMesh(core_axis_name="c", subcore_axis_name="s")


def _iota16():
    return lax.iota(jnp.int32, 16)


def _fill_rows(ref, nrows, ncols, value):
    iota = _iota16()
    vec = jnp.full((16,), value, jnp.float32)

    def body(r, _):
        rr = jnp.full((16,), r, jnp.int32)
        for cb in range(ncols // 16):
            plsc.store_scatter(ref, [rr, iota + cb * 16], vec)
        return 0

    lax.fori_loop(0, nrows, body, 0)


def _sc_counts_body(p_hbm, idx_out, inv_out, cnt_tbl, pbuf, obuf, ibuf, cbuf,
                    invb):
    c = lax.axis_index("c")
    s = lax.axis_index("s")
    iota = _iota16()
    zeros_i = jnp.zeros((16,), jnp.int32)

    _fill_rows(obuf, 128, 16, 1.0)

    for k in range(NC):
        b = c * NC + k
        base = pl.multiple_of(s * RPT, 8)

        _fill_rows(cbuf, RPT, 16, 0.0)
        pltpu.sync_copy(cbuf, cnt_tbl.at[pl.ds(base, RPT)])

        @pl.when(s == NS - 1)
        def _():
            pltpu.sync_copy(cbuf.at[pl.ds(0, TROWS - D3)],
                            cnt_tbl.at[pl.ds(D3, TROWS - D3)])

        plsc.subcore_barrier()

        def chunk(ch, _):
            off = pl.multiple_of(s * TPB + ch * 128, 8)
            pltpu.sync_copy(p_hbm.at[b, pl.ds(off, 128), :], pbuf)
            for g in range(8):
                rows = jnp.full((16,), g * 16, jnp.int32) + iota

                def coord(col):
                    v = plsc.load_gather(
                        pbuf, [rows, jnp.full((16,), col, jnp.int32)])
                    ncv = jnp.clip(v / DEN + 0.5, 0.0, HI)
                    return (ncv * float(RESO)).astype(jnp.int32)

                qx, qy, qz = coord(0), coord(1), coord(2)
                idxv = qx + RESO * (qy + RESO * qz)
                pos = jnp.full((16,), off + g * 16, jnp.int32) + iota
                idxv = jnp.where(pos < NPTS, idxv, D3)
                ibuf[pl.ds(g * 16, 16)] = idxv
            pltpu.sync_copy(obuf, cnt_tbl.at[ibuf], add=True)
            pltpu.sync_copy(ibuf, idx_out.at[b, s, ch])
            return 0

        lax.fori_loop(0, CH, chunk, 0)
        plsc.subcore_barrier()

        pltpu.sync_copy(cnt_tbl.at[pl.ds(base, RPT)], cbuf)

        def invrow(i, _):
            rows = jnp.full((16,), i * 16, jnp.int32) + iota
            cv = plsc.load_gather(cbuf, [rows, zeros_i])
            invb[pl.ds(i * 16, 16)] = 1.0 / jnp.maximum(cv, 1.0)
            return 0

        lax.fori_loop(0, RPT // 16, invrow, 0)
        pltpu.sync_copy(invb, inv_out.at[b, pl.ds(base, RPT)])
        plsc.subcore_barrier()


def _sc_pool_body(p1_hbm, idx_hbm, inv_hbm, pooled_out, tbl_out, ftbl,
                  rbuf, nbuf, ibuf, invb):
    c = lax.axis_index("c")
    s = lax.axis_index("s")
    iota = _iota16()

    for k in range(NC):
        b = c * NC + k
        base = pl.multiple_of(s * RPT, 8)

        _fill_rows(nbuf, 256, 32, 0.0)
        for zblk in range(RPT // 256):
            pltpu.sync_copy(nbuf, ftbl.at[pl.ds(base + zblk * 256, 256)])

        @pl.when(s == NS - 1)
        def _():
            pltpu.sync_copy(nbuf.at[pl.ds(0, TROWS - D3)],
                            ftbl.at[pl.ds(D3, TROWS - D3)])

        pltpu.sync_copy(idx_hbm.at[b, s], ibuf)
        pltpu.sync_copy(inv_hbm.at[b, pl.ds(base, RPT)], invb)
        plsc.subcore_barrier()

        def scatter(ch, _):
            off = pl.multiple_of(s * TPB + ch * 128, 8)
            pltpu.sync_copy(p1_hbm.at[b, pl.ds(off, 128), :], rbuf)
            pltpu.sync_copy(rbuf, ftbl.at[ibuf.at[ch]], add=True)
            return 0

        lax.fori_loop(0, CH, scatter, 0)
        plsc.subcore_barrier()

        for nb in range(RPT // 256):
            pltpu.sync_copy(ftbl.at[pl.ds(base + nb * 256, 256)], nbuf)

            def norm(i, _, _nb=nb):
                ii = jnp.full((16,), i, jnp.int32)
                spl = plsc.load_gather(invb, [jnp.full((16,), _nb * 256,
                                                       jnp.int32) + ii])
                r0 = plsc.load_gather(nbuf, [ii, iota])
                r1 = plsc.load_gather(nbuf, [ii, iota + 16])
                plsc.store_scatter(nbuf, [ii, iota], r0 * spl)
                plsc.store_scatter(nbuf, [ii, iota + 16], r1 * spl)
                return 0

            lax.fori_loop(0, 256, norm, 0)
            pltpu.sync_copy(nbuf, tbl_out.at[b, pl.ds(base + nb * 256, 256), :])
            pltpu.sync_copy(nbuf, ftbl.at[pl.ds(base + nb * 256, 256)])
        plsc.subcore_barrier()

        def gather(ch, _):
            off = pl.multiple_of(s * TPB + ch * 128, 8)
            pltpu.sync_copy(ftbl.at[ibuf.at[ch]], rbuf)
            pltpu.sync_copy(rbuf, pooled_out.at[b, pl.ds(off, 128), :])
            return 0

        lax.fori_loop(0, CH, gather, 0)
        plsc.subcore_barrier()


def _sc_counts(p_pad):
    return pl.kernel(
        _sc_counts_body,
        out_type=(jax.ShapeDtypeStruct((BS, NS, CH, 128), jnp.int32),
                  jax.ShapeDtypeStruct((BS, D3), jnp.float32)),
        mesh=_mesh(),
        compiler_params=pltpu.CompilerParams(needs_layout_passes=False, use_tc_tiling_on_sc=False),
        scratch_types=[
            pltpu.VMEM_SHARED((TROWS, 16), jnp.float32),
            pltpu.VMEM((128, 3), jnp.float32),
            pltpu.VMEM((128, 16), jnp.float32),
            pltpu.VMEM((128,), jnp.int32),
            pltpu.VMEM((RPT, 16), jnp.float32),
            pltpu.VMEM((RPT,), jnp.float32),
        ],
    )(p_pad)


def _sc_pool(p1, idx, inv):
    return pl.kernel(
        _sc_pool_body,
        out_type=(jax.ShapeDtypeStruct((BS, NP, MID), jnp.float32),
                  jax.ShapeDtypeStruct((BS, D3, MID), jnp.float32)),
        mesh=_mesh(),
        compiler_params=pltpu.CompilerParams(needs_layout_passes=False, use_tc_tiling_on_sc=False),
        scratch_types=[
            pltpu.VMEM_SHARED((TROWS, MID), jnp.float32),
            pltpu.VMEM((128, MID), jnp.float32),
            pltpu.VMEM((256, MID), jnp.float32),
            pltpu.VMEM((CH, 128), jnp.int32),
            pltpu.VMEM((RPT,), jnp.float32),
        ],
    )(p1, idx, inv)


def _res(x, wa, ba, wb, bb, wc):
    h = jnp.maximum(x, 0.0)
    h = jnp.dot(h, wa, preferred_element_type=jnp.float32) + ba
    h = jnp.maximum(h, 0.0)
    h = jnp.dot(h, wb, preferred_element_type=jnp.float32) + bb
    return jnp.dot(x, wc, preferred_element_type=jnp.float32) + h


def _tc_init_body(pf, w1e, b1, wa, ba, wb, bb, wc, o):
    x = jnp.dot(pf[...], w1e[...], preferred_element_type=jnp.float32) + b1[...]
    o[...] = _res(x, wa[...], ba[...], wb[...], bb[...], wc[...])


def _tc_res_body(pa, pb, wa, ba, wb, bb, wc, o):
    x = jnp.concatenate([pa[...], pb[...]], axis=1)
    o[...] = _res(x, wa[...], ba[...], wb[...], bb[...], wc[...])


def _tc_res_final_body(pa, pb, wa, ba, wb, bb, wc, wf, bf, o):
    x = jnp.concatenate([pa[...], pb[...]], axis=1)
    y = _res(x, wa[...], ba[...], wb[...], bb[...], wc[...])
    o[...] = jnp.dot(y, wf[...], preferred_element_type=jnp.float32) + bf[...]


def _tc_transpose_body(x, o):
    o[...] = jnp.transpose(x[...], (0, 2, 1))


def _tc_transpose(tbl):
    return pl.pallas_call(
        _tc_transpose_body,
        grid=(BS, D3 // 512),
        in_specs=[pl.BlockSpec((1, 512, MID), lambda b, j: (b, j, 0))],
        out_specs=pl.BlockSpec((1, MID, 512), lambda b, j: (b, 0, j)),
        out_shape=jax.ShapeDtypeStruct((BS, MID, D3), jnp.float32),
        compiler_params=pltpu.CompilerParams(
            dimension_semantics=("parallel", "parallel")),
    )(tbl)


def _full_spec(arr):
    nd = arr.ndim
    return pl.BlockSpec(arr.shape, lambda i, _nd=nd: (0,) * _nd)


def _tc_call(body, row_in, extras, n_row_in=1):
    rows = row_in[0].shape[0]
    grid = (rows // RB,)
    in_specs = ([pl.BlockSpec((RB, a.shape[1]), lambda i: (i, 0))
                 for a in row_in]
                + [_full_spec(a) for a in extras])
    return pl.pallas_call(
        body,
        grid=grid,
        in_specs=in_specs,
        out_specs=pl.BlockSpec((RB, MID), lambda i: (i, 0)),
        out_shape=jax.ShapeDtypeStruct((rows, MID), jnp.float32),
        compiler_params=pltpu.CompilerParams(
            dimension_semantics=("parallel",)),
    )(*row_in, *extras)


def kernel(p, params):
    p_pad = jnp.pad(p, ((0, 0), (0, NP - NPTS), (0, 0)))
    idx, inv = _sc_counts(p_pad)

    pf = jnp.pad(p_pad, ((0, 0), (0, 0), (0, 5))).reshape(BS * NP, 8)
    w1e = jnp.pad(params['w1'].T, ((0, 5), (0, 0)))

    def blk(i):
        return (params['blk%d_fc1_w' % i].T, params['blk%d_fc1_b' % i][None],
                params['blk%d_fc2_w' % i].T, params['blk%d_fc2_b' % i][None],
                params['blk%d_fc3_w' % i].T)

    p1 = _tc_call(_tc_init_body, [pf],
                  [w1e, params['b1'][None]] + list(blk(0)))

    eye = jnp.eye(MID, dtype=jnp.float32)
    zb = jnp.zeros((1, MID), jnp.float32)
    zw = jnp.zeros((2 * MID, MID), jnp.float32)
    stages = [list(blk(i)) + [eye, zb] for i in range(1, 4)]
    stages.append(list(blk(4)) + [params['w2'].T, params['b2'][None]])
    stages.append([zw, zb, jnp.zeros((MID, MID), jnp.float32), zb, zw,
                   eye, zb])
    ws = tuple(jnp.stack([st[j] for st in stages]) for j in range(7))

    def step(carry, w):
        p1_c, _ = carry
        pooled, tbl = _sc_pool(p1_c.reshape(BS, NP, MID), idx, inv)
        p1_n = _tc_call(_tc_res_final_body,
                        [p1_c, pooled.reshape(BS * NP, MID)], list(w))
        return (p1_n, tbl), None

    (_, tbl), _ = lax.scan(
        step, (p1, jnp.zeros((BS, D3, MID), jnp.float32)), ws)
    return _tc_transpose(tbl).reshape(BS, MID, RESO, RESO, RESO)


# speedup vs baseline: 222.8168x; 1.0912x over previous
import functools

import jax
import jax.numpy as jnp
from jax import lax
from jax.experimental import pallas as pl
from jax.experimental.pallas import tpu as pltpu
from jax.experimental.pallas import tpu_sc as plsc

BS = 4
NPTS = 100000
MID = 32
RESO = 32
D3 = RESO ** 3
NC = 2
NS = 16
CH = 49
TPB = CH * 128
NP = NS * TPB
TROWS = D3 + 16
RPT = D3 // NS
DEN = 1.0 + 0.1 + 0.001
HI = 1.0 - 0.001
RB = 1024


def _mesh():
    return plsc.VectorSubcoreMesh(core_axis_name="c", subcore_axis_name="s")


def _iota16():
    return lax.iota(jnp.int32, 16)


def _fill_rows(ref, nrows, ncols, value):
    iota = _iota16()
    vec = jnp.full((16,), value, jnp.float32)

    def body(r, _):
        rr = jnp.full((16,), r, jnp.int32)
        for cb in range(ncols // 16):
            plsc.store_scatter(ref, [rr, iota + cb * 16], vec)
        return 0

    lax.fori_loop(0, nrows, body, 0)


def _sc_counts_body(p_hbm, idx_out, inv_out, cnt_tbl, pbuf, obuf, ibuf, cbuf,
                    invb):
    c = lax.axis_index("c")
    s = lax.axis_index("s")
    iota = _iota16()
    zeros_i = jnp.zeros((16,), jnp.int32)

    _fill_rows(obuf, 128, 16, 1.0)

    for k in range(NC):
        b = c * NC + k
        base = pl.multiple_of(s * RPT, 8)

        _fill_rows(cbuf, RPT, 16, 0.0)
        pltpu.sync_copy(cbuf, cnt_tbl.at[pl.ds(base, RPT)])

        @pl.when(s == NS - 1)
        def _():
            pltpu.sync_copy(cbuf.at[pl.ds(0, TROWS - D3)],
                            cnt_tbl.at[pl.ds(D3, TROWS - D3)])

        plsc.subcore_barrier()

        def chunk(ch, _):
            off = pl.multiple_of(s * TPB + ch * 128, 8)
            pltpu.sync_copy(p_hbm.at[b, pl.ds(off, 128), :], pbuf)
            for g in range(8):
                rows = jnp.full((16,), g * 16, jnp.int32) + iota

                def coord(col):
                    v = plsc.load_gather(
                        pbuf, [rows, jnp.full((16,), col, jnp.int32)])
                    ncv = jnp.clip(v / DEN + 0.5, 0.0, HI)
                    return (ncv * float(RESO)).astype(jnp.int32)

                qx, qy, qz = coord(0), coord(1), coord(2)
                idxv = qx + RESO * (qy + RESO * qz)
                pos = jnp.full((16,), off + g * 16, jnp.int32) + iota
                idxv = jnp.where(pos < NPTS, idxv, D3)
                ibuf[pl.ds(g * 16, 16)] = idxv
            pltpu.sync_copy(obuf, cnt_tbl.at[ibuf], add=True)
            pltpu.sync_copy(ibuf, idx_out.at[b, s, ch])
            return 0

        lax.fori_loop(0, CH, chunk, 0)
        plsc.subcore_barrier()

        pltpu.sync_copy(cnt_tbl.at[pl.ds(base, RPT)], cbuf)

        def invrow(i, _):
            rows = jnp.full((16,), i * 16, jnp.int32) + iota
            cv = plsc.load_gather(cbuf, [rows, zeros_i])
            invb[pl.ds(i * 16, 16)] = 1.0 / jnp.maximum(cv, 1.0)
            return 0

        lax.fori_loop(0, RPT // 16, invrow, 0)
        pltpu.sync_copy(invb, inv_out.at[b, pl.ds(base, RPT)])
        plsc.subcore_barrier()


def _sc_pool_body(p1_hbm, idx_hbm, inv_hbm, pooled_out, tbl_out, ftbl,
                  rbuf, nbuf, ibuf, invb):
    c = lax.axis_index("c")
    s = lax.axis_index("s")
    iota = _iota16()

    for k in range(NC):
        b = c * NC + k
        base = pl.multiple_of(s * RPT, 8)

        _fill_rows(nbuf, 256, 32, 0.0)
        for zblk in range(RPT // 256):
            pltpu.sync_copy(nbuf, ftbl.at[pl.ds(base + zblk * 256, 256)])

        @pl.when(s == NS - 1)
        def _():
            pltpu.sync_copy(nbuf.at[pl.ds(0, TROWS - D3)],
                            ftbl.at[pl.ds(D3, TROWS - D3)])

        pltpu.sync_copy(idx_hbm.at[b, s], ibuf)
        pltpu.sync_copy(inv_hbm.at[b, pl.ds(base, RPT)], invb)
        plsc.subcore_barrier()

        def scatter(ch, _):
            off = pl.multiple_of(b * NP + s * TPB + ch * 128, 8)
            pltpu.sync_copy(p1_hbm.at[pl.ds(off, 128), :], rbuf)
            pltpu.sync_copy(rbuf, ftbl.at[ibuf.at[ch]], add=True)
            return 0

        lax.fori_loop(0, CH, scatter, 0)
        plsc.subcore_barrier()

        for nb in range(RPT // 256):
            pltpu.sync_copy(ftbl.at[pl.ds(base + nb * 256, 256)], nbuf)

            def norm(i, _, _nb=nb):
                ii = jnp.full((16,), i, jnp.int32)
                spl = plsc.load_gather(invb, [jnp.full((16,), _nb * 256,
                                                       jnp.int32) + ii])
                r0 = plsc.load_gather(nbuf, [ii, iota])
                r1 = plsc.load_gather(nbuf, [ii, iota + 16])
                plsc.store_scatter(nbuf, [ii, iota], r0 * spl)
                plsc.store_scatter(nbuf, [ii, iota + 16], r1 * spl)
                return 0

            lax.fori_loop(0, 256, norm, 0)
            pltpu.sync_copy(nbuf, tbl_out.at[b, pl.ds(base + nb * 256, 256), :])
            pltpu.sync_copy(nbuf, ftbl.at[pl.ds(base + nb * 256, 256)])
        plsc.subcore_barrier()

        def gather(ch, _):
            off = pl.multiple_of(b * NP + s * TPB + ch * 128, 8)
            pltpu.sync_copy(ftbl.at[ibuf.at[ch]], rbuf)
            pltpu.sync_copy(rbuf, pooled_out.at[pl.ds(off, 128), :])
            return 0

        lax.fori_loop(0, CH, gather, 0)
        plsc.subcore_barrier()


def _sc_counts(p_pad):
    return pl.kernel(
        _sc_counts_body,
        out_type=(jax.ShapeDtypeStruct((BS, NS, CH, 128), jnp.int32),
                  jax.ShapeDtypeStruct((BS, D3), jnp.float32)),
        mesh=_mesh(),
        compiler_params=pltpu.CompilerParams(needs_layout_passes=False, use_tc_tiling_on_sc=False),
        scratch_types=[
            pltpu.VMEM_SHARED((TROWS, 16), jnp.float32),
            pltpu.VMEM((128, 3), jnp.float32),
            pltpu.VMEM((128, 16), jnp.float32),
            pltpu.VMEM((128,), jnp.int32),
            pltpu.VMEM((RPT, 16), jnp.float32),
            pltpu.VMEM((RPT,), jnp.float32),
        ],
    )(p_pad)


def _sc_pool(p1, idx, inv):
    return pl.kernel(
        _sc_pool_body,
        out_type=(jax.ShapeDtypeStruct((BS * NP, MID), jnp.float32),
                  jax.ShapeDtypeStruct((BS, D3, MID), jnp.float32)),
        mesh=_mesh(),
        compiler_params=pltpu.CompilerParams(needs_layout_passes=False, use_tc_tiling_on_sc=False),
        scratch_types=[
            pltpu.VMEM_SHARED((TROWS, MID), jnp.float32),
            pltpu.VMEM((128, MID), jnp.float32),
            pltpu.VMEM((256, MID), jnp.float32),
            pltpu.VMEM((CH, 128), jnp.int32),
            pltpu.VMEM((RPT,), jnp.float32),
        ],
    )(p1, idx, inv)


def _res(x, wa, ba, wb, bb, wc):
    h = jnp.maximum(x, 0.0)
    h = jnp.dot(h, wa, preferred_element_type=jnp.float32) + ba
    h = jnp.maximum(h, 0.0)
    h = jnp.dot(h, wb, preferred_element_type=jnp.float32) + bb
    return jnp.dot(x, wc, preferred_element_type=jnp.float32) + h


def _tc_init_body(pf, w1e, b1, wa, ba, wb, bb, wc, o):
    x = (jnp.dot(pf[0], w1e[...], preferred_element_type=jnp.float32)
         + b1[...])
    o[...] = _res(x, wa[...], ba[...], wb[...], bb[...], wc[...])


def _tc_init(p_pad, w1t, b1, wa, ba, wb, bb, wc):
    extras = [w1t, b1, wa, ba, wb, bb, wc]
    return pl.pallas_call(
        _tc_init_body,
        grid=(BS * NP // RB,),
        in_specs=([pl.BlockSpec((1, RB, 3), lambda i: (i // (NP // RB),
                                                       i % (NP // RB), 0))]
                  + [_full_spec(a) for a in extras]),
        out_specs=pl.BlockSpec((RB, MID), lambda i: (i, 0)),
        out_shape=jax.ShapeDtypeStruct((BS * NP, MID), jnp.float32),
        compiler_params=pltpu.CompilerParams(
            dimension_semantics=("parallel",)),
    )(p_pad, *extras)


def _tc_res_final_body(pa, pb, wa, ba, wb, bb, wc, wf, bf, o):
    x = jnp.concatenate([pa[...], pb[...]], axis=1)
    y = _res(x, wa[...], ba[...], wb[...], bb[...], wc[...])
    o[...] = jnp.dot(y, wf[...], preferred_element_type=jnp.float32) + bf[...]


def _tc_transpose_body(x, o):
    o[...] = jnp.transpose(x[...], (0, 2, 1))


def _tc_transpose(tbl):
    return pl.pallas_call(
        _tc_transpose_body,
        grid=(BS, D3 // 512),
        in_specs=[pl.BlockSpec((1, 512, MID), lambda b, j: (b, j, 0))],
        out_specs=pl.BlockSpec((1, MID, 512), lambda b, j: (b, 0, j)),
        out_shape=jax.ShapeDtypeStruct((BS, MID, D3), jnp.float32),
        compiler_params=pltpu.CompilerParams(
            dimension_semantics=("parallel", "parallel")),
    )(tbl)


def _full_spec(arr):
    nd = arr.ndim
    return pl.BlockSpec(arr.shape, lambda i, _nd=nd: (0,) * _nd)


def _tc_call(body, row_in, extras, alias_first=False):
    rows = row_in[0].shape[0]
    grid = (rows // RB,)
    in_specs = ([pl.BlockSpec((RB, a.shape[1]), lambda i: (i, 0))
                 for a in row_in]
                + [_full_spec(a) for a in extras])
    return pl.pallas_call(
        body,
        grid=grid,
        in_specs=in_specs,
        out_specs=pl.BlockSpec((RB, MID), lambda i: (i, 0)),
        out_shape=jax.ShapeDtypeStruct((rows, MID), jnp.float32),
        input_output_aliases={0: 0} if alias_first else {},
        compiler_params=pltpu.CompilerParams(
            dimension_semantics=("parallel",)),
    )(*row_in, *extras)


def kernel(p, params):
    p_pad = jnp.pad(p, ((0, 0), (0, NP - NPTS), (0, 0)))
    idx, inv = _sc_counts(p_pad)

    def blk(i):
        return (params['blk%d_fc1_w' % i].T, params['blk%d_fc1_b' % i][None],
                params['blk%d_fc2_w' % i].T, params['blk%d_fc2_b' % i][None],
                params['blk%d_fc3_w' % i].T)

    p1 = _tc_init(p_pad, params['w1'].T, params['b1'][None], *blk(0))

    eye = jnp.eye(MID, dtype=jnp.float32)
    zb = jnp.zeros((1, MID), jnp.float32)
    zw = jnp.zeros((2 * MID, MID), jnp.float32)
    stages = [list(blk(i)) + [eye, zb] for i in range(1, 4)]
    stages.append(list(blk(4)) + [params['w2'].T, params['b2'][None]])
    stages.append([zw, zb, jnp.zeros((MID, MID), jnp.float32), zb, zw,
                   eye, zb])
    ws = tuple(jnp.stack([st[j] for st in stages]) for j in range(7))

    def step(carry, w):
        p1_c, _ = carry
        pooled, tbl = _sc_pool(p1_c, idx, inv)
        p1_n = _tc_call(_tc_res_final_body, [p1_c, pooled], list(w),
                        alias_first=True)
        return (p1_n, tbl), None

    (_, tbl), _ = lax.scan(
        step, (p1, jnp.zeros((BS, D3, MID), jnp.float32)), ws)
    return _tc_transpose(tbl).reshape(BS, MID, RESO, RESO, RESO)


# speedup vs baseline: 320.7377x; 1.4395x over previous
import functools

import jax
import jax.numpy as jnp
from jax import lax
from jax.experimental import pallas as pl
from jax.experimental.pallas import tpu as pltpu
from jax.experimental.pallas import tpu_sc as plsc

BS = 4
NPTS = 100000
MID = 32
RESO = 32
D3 = RESO ** 3
NC = 2
NS = 16
CH = 49
TPB = CH * 128
NP = NS * TPB
TROWS = D3 + 16
RPT = D3 // NS
DEN = 1.0 + 0.1 + 0.001
HI = 1.0 - 0.001
RB = 1024


def _mesh():
    return plsc.VectorSubcoreMesh(core_axis_name="c", subcore_axis_name="s")


def _iota16():
    return lax.iota(jnp.int32, 16)


def _fill_rows(ref, nrows, ncols, value):
    iota = _iota16()
    vec = jnp.full((16,), value, jnp.float32)

    def body(r, _):
        rr = jnp.full((16,), r, jnp.int32)
        for cb in range(ncols // 16):
            plsc.store_scatter(ref, [rr, iota + cb * 16], vec)
        return 0

    lax.fori_loop(0, nrows, body, 0)


def _sc_counts_body(p_hbm, idx_out, inv_out, cnt_tbl, pbuf, obuf, ibuf, cbuf,
                    invb):
    c = lax.axis_index("c")
    s = lax.axis_index("s")
    iota = _iota16()
    zeros_i = jnp.zeros((16,), jnp.int32)

    _fill_rows(obuf, 128, 16, 1.0)

    for k in range(NC):
        b = c * NC + k
        base = pl.multiple_of(s * RPT, 8)

        _fill_rows(cbuf, RPT, 16, 0.0)
        pltpu.sync_copy(cbuf, cnt_tbl.at[pl.ds(base, RPT)])

        @pl.when(s == NS - 1)
        def _():
            pltpu.sync_copy(cbuf.at[pl.ds(0, TROWS - D3)],
                            cnt_tbl.at[pl.ds(D3, TROWS - D3)])

        plsc.subcore_barrier()

        def chunk(ch, _):
            off = pl.multiple_of(s * TPB + ch * 128, 8)
            pltpu.sync_copy(p_hbm.at[b, pl.ds(off, 128), :], pbuf)
            for g in range(8):
                rows = jnp.full((16,), g * 16, jnp.int32) + iota

                def coord(col):
                    v = plsc.load_gather(
                        pbuf, [rows, jnp.full((16,), col, jnp.int32)])
                    ncv = jnp.clip(v / DEN + 0.5, 0.0, HI)
                    return (ncv * float(RESO)).astype(jnp.int32)

                qx, qy, qz = coord(0), coord(1), coord(2)
                idxv = qx + RESO * (qy + RESO * qz)
                pos = jnp.full((16,), off + g * 16, jnp.int32) + iota
                idxv = jnp.where(pos < NPTS, idxv, D3)
                ibuf[pl.ds(g * 16, 16)] = idxv
            pltpu.sync_copy(obuf, cnt_tbl.at[ibuf], add=True)
            pltpu.sync_copy(ibuf, idx_out.at[b, s, ch])
            return 0

        lax.fori_loop(0, CH, chunk, 0)
        plsc.subcore_barrier()

        pltpu.sync_copy(cnt_tbl.at[pl.ds(base, RPT)], cbuf)

        def invrow(i, _):
            rows = jnp.full((16,), i * 16, jnp.int32) + iota
            cv = plsc.load_gather(cbuf, [rows, zeros_i])
            invb[pl.ds(i * 16, 16)] = 1.0 / jnp.maximum(cv, 1.0)
            return 0

        lax.fori_loop(0, RPT // 16, invrow, 0)
        pltpu.sync_copy(invb, inv_out.at[b, pl.ds(base, RPT)])
        plsc.subcore_barrier()


def _sc_pool_body(p1_hbm, idx_hbm, inv_hbm, pooled_out, tbl_out, ftbl,
                  rbuf, nbuf, ibuf, invb):
    c = lax.axis_index("c")
    s = lax.axis_index("s")
    iota = _iota16()

    for k in range(NC):
        b = c * NC + k
        base = pl.multiple_of(s * RPT, 8)

        _fill_rows(nbuf, 256, 32, 0.0)
        for zblk in range(RPT // 256):
            pltpu.sync_copy(nbuf, ftbl.at[pl.ds(base + zblk * 256, 256)])

        @pl.when(s == NS - 1)
        def _():
            pltpu.sync_copy(nbuf.at[pl.ds(0, TROWS - D3)],
                            ftbl.at[pl.ds(D3, TROWS - D3)])

        pltpu.sync_copy(idx_hbm.at[b, s], ibuf)
        pltpu.sync_copy(inv_hbm.at[b, pl.ds(base, RPT)], invb)
        plsc.subcore_barrier()

        def scatter(ch, _):
            off = pl.multiple_of(b * NP + s * TPB + ch * 128, 8)
            pltpu.sync_copy(p1_hbm.at[pl.ds(off, 128), :], rbuf)
            pltpu.sync_copy(rbuf, ftbl.at[ibuf.at[ch]], add=True)
            return 0

        lax.fori_loop(0, CH, scatter, 0)
        plsc.subcore_barrier()

        for nb in range(RPT // 256):
            pltpu.sync_copy(ftbl.at[pl.ds(base + nb * 256, 256)], nbuf)

            def norm(i, _, _nb=nb):
                ii = jnp.full((16,), i, jnp.int32)
                spl = plsc.load_gather(invb, [jnp.full((16,), _nb * 256,
                                                       jnp.int32) + ii])
                r0 = plsc.load_gather(nbuf, [ii, iota])
                r1 = plsc.load_gather(nbuf, [ii, iota + 16])
                plsc.store_scatter(nbuf, [ii, iota], r0 * spl)
                plsc.store_scatter(nbuf, [ii, iota + 16], r1 * spl)
                return 0

            lax.fori_loop(0, 256, norm, 0)
            pltpu.sync_copy(nbuf, tbl_out.at[b, pl.ds(base + nb * 256, 256), :])
            pltpu.sync_copy(nbuf, ftbl.at[pl.ds(base + nb * 256, 256)])
        plsc.subcore_barrier()

        def gather(ch, _):
            off = pl.multiple_of(b * NP + s * TPB + ch * 128, 8)
            pltpu.sync_copy(ftbl.at[ibuf.at[ch]], rbuf)
            pltpu.sync_copy(rbuf, pooled_out.at[pl.ds(off, 128), :])
            return 0

        lax.fori_loop(0, CH, gather, 0)
        plsc.subcore_barrier()


def _sc_counts(p_pad):
    return pl.kernel(
        _sc_counts_body,
        out_type=(jax.ShapeDtypeStruct((BS, NS, CH, 128), jnp.int32),
                  jax.ShapeDtypeStruct((BS, D3), jnp.float32)),
        mesh=_mesh(),
        compiler_params=pltpu.CompilerParams(needs_layout_passes=False, use_tc_tiling_on_sc=False),
        scratch_types=[
            pltpu.VMEM_SHARED((TROWS, 16), jnp.float32),
            pltpu.VMEM((128, 3), jnp.float32),
            pltpu.VMEM((128, 16), jnp.float32),
            pltpu.VMEM((128,), jnp.int32),
            pltpu.VMEM((RPT, 16), jnp.float32),
            pltpu.VMEM((RPT,), jnp.float32),
        ],
    )(p_pad)


def _sc_pool(p1, idx, inv):
    return pl.kernel(
        _sc_pool_body,
        out_type=(jax.ShapeDtypeStruct((BS * NP, MID), jnp.float32),
                  jax.ShapeDtypeStruct((BS, D3, MID), jnp.float32)),
        mesh=_mesh(),
        compiler_params=pltpu.CompilerParams(needs_layout_passes=False, use_tc_tiling_on_sc=False),
        scratch_types=[
            pltpu.VMEM_SHARED((TROWS, MID), jnp.float32),
            pltpu.VMEM((128, MID), jnp.float32),
            pltpu.VMEM((256, MID), jnp.float32),
            pltpu.VMEM((CH, 128), jnp.int32),
            pltpu.VMEM((RPT,), jnp.float32),
        ],
    )(p1, idx, inv)


def _res(x, wa, ba, wb, bb, wc):
    h = jnp.maximum(x, 0.0)
    h = jnp.dot(h, wa, preferred_element_type=jnp.float32) + ba
    h = jnp.maximum(h, 0.0)
    h = jnp.dot(h, wb, preferred_element_type=jnp.float32) + bb
    return jnp.dot(x, wc, preferred_element_type=jnp.float32) + h


def _tc_init_body(pf, w1e, b1, wa, ba, wb, bb, wc, o):
    x = (jnp.dot(pf[0], w1e[...], preferred_element_type=jnp.float32)
         + b1[...])
    o[...] = _res(x, wa[...], ba[...], wb[...], bb[...], wc[...])


def _tc_init(p_pad, w1t, b1, wa, ba, wb, bb, wc):
    extras = [w1t, b1, wa, ba, wb, bb, wc]
    return pl.pallas_call(
        _tc_init_body,
        grid=(BS * NP // RB,),
        in_specs=([pl.BlockSpec((1, RB, 3), lambda i: (i // (NP // RB),
                                                       i % (NP // RB), 0))]
                  + [_full_spec(a) for a in extras]),
        out_specs=pl.BlockSpec((RB, MID), lambda i: (i, 0)),
        out_shape=jax.ShapeDtypeStruct((BS * NP, MID), jnp.float32),
        compiler_params=pltpu.CompilerParams(
            dimension_semantics=("parallel",)),
    )(p_pad, *extras)


def _tc_res_packed_body(pa, pb, at, ab, ba, wb, bb, ct, cb, wf, bf, o):
    xa, xb = pa[...], pb[...]
    ra = jnp.maximum(xa, 0.0)
    rb = jnp.maximum(xb, 0.0)
    dot = lambda x, w: jnp.dot(x, w, preferred_element_type=jnp.float32)
    h = dot(ra, at[...]) + dot(rb, ab[...]) + ba[...]
    h = dot(jnp.maximum(h, 0.0), wb[...]) + bb[...]
    y = dot(xa, ct[...]) + dot(xb, cb[...]) + h
    o[...] = dot(y, wf[...]) + bf[...]


def _tc_transpose_body(x, o):
    o[...] = jnp.transpose(x[...], (0, 2, 1))


def _tc_transpose(tbl):
    return pl.pallas_call(
        _tc_transpose_body,
        grid=(BS, D3 // 512),
        in_specs=[pl.BlockSpec((1, 512, MID), lambda b, j: (b, j, 0))],
        out_specs=pl.BlockSpec((1, MID, 512), lambda b, j: (b, 0, j)),
        out_shape=jax.ShapeDtypeStruct((BS, MID, D3), jnp.float32),
        compiler_params=pltpu.CompilerParams(
            dimension_semantics=("parallel", "parallel")),
    )(tbl)


def _full_spec(arr):
    nd = arr.ndim
    return pl.BlockSpec(arr.shape, lambda i, _nd=nd: (0,) * _nd)


def _tc_res_packed(p1p, pooledp, w):
    rows = p1p.shape[0]
    rb = RB // 4
    in_specs = ([pl.BlockSpec((rb, 128), lambda i: (i, 0))] * 2
                + [_full_spec(a) for a in w])
    return pl.pallas_call(
        _tc_res_packed_body,
        grid=(rows // rb,),
        in_specs=in_specs,
        out_specs=pl.BlockSpec((rb, 128), lambda i: (i, 0)),
        out_shape=jax.ShapeDtypeStruct((rows, 128), jnp.float32),
        input_output_aliases={0: 0},
        compiler_params=pltpu.CompilerParams(
            dimension_semantics=("parallel",)),
    )(p1p, pooledp, *w)


def kernel(p, params):
    p_pad = jnp.pad(p, ((0, 0), (0, NP - NPTS), (0, 0)))
    idx, inv = _sc_counts(p_pad)

    def blk(i):
        return (params['blk%d_fc1_w' % i].T, params['blk%d_fc1_b' % i][None],
                params['blk%d_fc2_w' % i].T, params['blk%d_fc2_b' % i][None],
                params['blk%d_fc3_w' % i].T)

    p1 = _tc_init(p_pad, params['w1'].T, params['b1'][None], *blk(0))

    eye4 = jnp.eye(4, dtype=jnp.float32)
    bd = lambda w: jnp.kron(eye4, w)
    t4 = lambda b: jnp.tile(b, (1, 4))

    def pack_stage(wa, ba, wb, bb, wc, wf, bf):
        return (bd(wa[:MID]), bd(wa[MID:]), t4(ba), bd(wb), t4(bb),
                bd(wc[:MID]), bd(wc[MID:]), bd(wf), t4(bf))

    eye = jnp.eye(MID, dtype=jnp.float32)
    zb = jnp.zeros((1, MID), jnp.float32)
    zw = jnp.zeros((2 * MID, MID), jnp.float32)
    stages = [pack_stage(*blk(i), eye, zb) for i in range(1, 4)]
    stages.append(pack_stage(*blk(4), params['w2'].T, params['b2'][None]))
    stages.append(pack_stage(zw, zb, jnp.zeros((MID, MID), jnp.float32),
                             zb, zw, eye, zb))
    ws = tuple(jnp.stack([st[j] for st in stages]) for j in range(9))

    def step(carry, w):
        p1_c, _ = carry
        pooled, tbl = _sc_pool(jnp.reshape(p1_c, (BS * NP, MID)), idx, inv)
        pooledp = jnp.reshape(pooled, (BS * NP // 4, 128))
        p1_n = _tc_res_packed(p1_c, pooledp, list(w))
        return (p1_n, tbl), None

    p1p = jnp.reshape(p1, (BS * NP // 4, 128))
    (_, tbl), _ = lax.scan(
        step, (p1p, jnp.zeros((BS, D3, MID), jnp.float32)), ws)
    return _tc_transpose(tbl).reshape(BS, MID, RESO, RESO, RESO)


# speedup vs baseline: 381.4242x; 1.1892x over previous
import functools

import jax
import jax.numpy as jnp
from jax import lax
from jax.experimental import pallas as pl
from jax.experimental.pallas import tpu as pltpu
from jax.experimental.pallas import tpu_sc as plsc

BS = 4
NPTS = 100000
MID = 32
RESO = 32
D3 = RESO ** 3
NC = 2
NS = 16
CH = 49
TPB = CH * 128
NP = NS * TPB
TROWS = D3 + 16
RPT = D3 // NS
DEN = 1.0 + 0.1 + 0.001
HI = 1.0 - 0.001
RB = 1024


def _mesh():
    return plsc.VectorSubcoreMesh(core_axis_name="c", subcore_axis_name="s")


def _iota16():
    return lax.iota(jnp.int32, 16)


def _fill_rows(ref, nrows, ncols, value):
    iota = _iota16()
    vec = jnp.full((16,), value, jnp.float32)

    def body(r, _):
        rr = jnp.full((16,), r, jnp.int32)
        for cb in range(ncols // 16):
            plsc.store_scatter(ref, [rr, iota + cb * 16], vec)
        return 0

    lax.fori_loop(0, nrows, body, 0)


def _sc_counts_body(p_hbm, idx_out, inv_out, cnt_tbl, pbuf, obuf, ibuf, cbuf,
                    invb):
    c = lax.axis_index("c")
    s = lax.axis_index("s")
    iota = _iota16()
    zeros_i = jnp.zeros((16,), jnp.int32)

    _fill_rows(obuf, 128, 16, 1.0)

    for k in range(NC):
        b = c * NC + k
        base = pl.multiple_of(s * RPT, 8)

        _fill_rows(cbuf, RPT, 16, 0.0)
        pltpu.sync_copy(cbuf, cnt_tbl.at[pl.ds(base, RPT)])

        @pl.when(s == NS - 1)
        def _():
            pltpu.sync_copy(cbuf.at[pl.ds(0, TROWS - D3)],
                            cnt_tbl.at[pl.ds(D3, TROWS - D3)])

        plsc.subcore_barrier()

        def chunk(ch, _):
            off = pl.multiple_of(s * TPB + ch * 128, 8)
            pltpu.sync_copy(p_hbm.at[b, pl.ds(off, 128), :], pbuf)
            for g in range(8):
                rows = jnp.full((16,), g * 16, jnp.int32) + iota

                def coord(col):
                    v = plsc.load_gather(
                        pbuf, [rows, jnp.full((16,), col, jnp.int32)])
                    ncv = jnp.clip(v / DEN + 0.5, 0.0, HI)
                    return (ncv * float(RESO)).astype(jnp.int32)

                qx, qy, qz = coord(0), coord(1), coord(2)
                idxv = qx + RESO * (qy + RESO * qz)
                pos = jnp.full((16,), off + g * 16, jnp.int32) + iota
                idxv = jnp.where(pos < NPTS, idxv, D3)
                ibuf[pl.ds(g * 16, 16)] = idxv
            pltpu.sync_copy(obuf, cnt_tbl.at[ibuf], add=True)
            pltpu.sync_copy(ibuf, idx_out.at[b, s, ch])
            return 0

        lax.fori_loop(0, CH, chunk, 0)
        plsc.subcore_barrier()

        pltpu.sync_copy(cnt_tbl.at[pl.ds(base, RPT)], cbuf)

        def invrow(i, _):
            rows = jnp.full((16,), i * 16, jnp.int32) + iota
            cv = plsc.load_gather(cbuf, [rows, zeros_i])
            invb[pl.ds(i * 16, 16)] = 1.0 / jnp.maximum(cv, 1.0)
            return 0

        lax.fori_loop(0, RPT // 16, invrow, 0)
        pltpu.sync_copy(invb, inv_out.at[b, pl.ds(base, RPT)])
        plsc.subcore_barrier()


def _sc_pool_body(p1_hbm, idx_hbm, inv_hbm, pooled_out, tbl_out, ftbl,
                  rbuf, nbuf, ibuf, invb):
    c = lax.axis_index("c")
    s = lax.axis_index("s")
    iota = _iota16()

    for k in range(NC):
        b = c * NC + k
        base = pl.multiple_of(s * RPT, 8)

        _fill_rows(nbuf, 256, 32, 0.0)
        for zblk in range(RPT // 256):
            pltpu.sync_copy(nbuf, ftbl.at[pl.ds(base + zblk * 256, 256)])

        @pl.when(s == NS - 1)
        def _():
            pltpu.sync_copy(nbuf.at[pl.ds(0, TROWS - D3)],
                            ftbl.at[pl.ds(D3, TROWS - D3)])

        pltpu.sync_copy(idx_hbm.at[b, s], ibuf)
        pltpu.sync_copy(inv_hbm.at[b, pl.ds(base, RPT)], invb)
        plsc.subcore_barrier()

        def scatter(ch, _):
            off = pl.multiple_of(b * NP + s * TPB + ch * 128, 8)
            pltpu.sync_copy(p1_hbm.at[pl.ds(off, 128), :], rbuf)
            pltpu.sync_copy(rbuf, ftbl.at[ibuf.at[ch]], add=True)
            return 0

        lax.fori_loop(0, CH, scatter, 0)
        plsc.subcore_barrier()

        for nb in range(RPT // 256):
            pltpu.sync_copy(ftbl.at[pl.ds(base + nb * 256, 256)], nbuf)

            def norm(i, _, _nb=nb):
                ii = jnp.full((16,), i, jnp.int32)
                spl = plsc.load_gather(invb, [jnp.full((16,), _nb * 256,
                                                       jnp.int32) + ii])
                r0 = plsc.load_gather(nbuf, [ii, iota])
                r1 = plsc.load_gather(nbuf, [ii, iota + 16])
                plsc.store_scatter(nbuf, [ii, iota], r0 * spl)
                plsc.store_scatter(nbuf, [ii, iota + 16], r1 * spl)
                return 0

            lax.fori_loop(0, 256, norm, 0)
            pltpu.sync_copy(nbuf, tbl_out.at[b, pl.ds(base + nb * 256, 256), :])
            pltpu.sync_copy(nbuf, ftbl.at[pl.ds(base + nb * 256, 256)])
        plsc.subcore_barrier()

        def gather(ch, _):
            off = pl.multiple_of(b * NP + s * TPB + ch * 128, 8)
            pltpu.sync_copy(ftbl.at[ibuf.at[ch]], rbuf)
            pltpu.sync_copy(rbuf, pooled_out.at[pl.ds(off, 128), :])
            return 0

        lax.fori_loop(0, CH, gather, 0)
        plsc.subcore_barrier()


def _sc_counts(p_pad):
    return pl.kernel(
        _sc_counts_body,
        out_type=(jax.ShapeDtypeStruct((BS, NS, CH, 128), jnp.int32),
                  jax.ShapeDtypeStruct((BS, D3), jnp.float32)),
        mesh=_mesh(),
        compiler_params=pltpu.CompilerParams(needs_layout_passes=False, use_tc_tiling_on_sc=False),
        scratch_types=[
            pltpu.VMEM_SHARED((TROWS, 16), jnp.float32),
            pltpu.VMEM((128, 3), jnp.float32),
            pltpu.VMEM((128, 16), jnp.float32),
            pltpu.VMEM((128,), jnp.int32),
            pltpu.VMEM((RPT, 16), jnp.float32),
            pltpu.VMEM((RPT,), jnp.float32),
        ],
    )(p_pad)


def _sc_pool(p1, idx, inv):
    return pl.kernel(
        _sc_pool_body,
        out_type=(jax.ShapeDtypeStruct((BS * NP, MID), jnp.float32),
                  jax.ShapeDtypeStruct((BS, D3, MID), jnp.float32)),
        mesh=_mesh(),
        compiler_params=pltpu.CompilerParams(needs_layout_passes=False, use_tc_tiling_on_sc=False),
        scratch_types=[
            pltpu.VMEM_SHARED((TROWS, MID), jnp.float32),
            pltpu.VMEM((128, MID), jnp.float32),
            pltpu.VMEM((256, MID), jnp.float32),
            pltpu.VMEM((CH, 128), jnp.int32),
            pltpu.VMEM((RPT,), jnp.float32),
        ],
    )(p1, idx, inv)


def _res(x, wa, ba, wb, bb, wc):
    h = jnp.maximum(x, 0.0)
    h = jnp.dot(h, wa, preferred_element_type=jnp.float32) + ba
    h = jnp.maximum(h, 0.0)
    h = jnp.dot(h, wb, preferred_element_type=jnp.float32) + bb
    return jnp.dot(x, wc, preferred_element_type=jnp.float32) + h


def _tc_init_body(pf, w1e, b1, wa, ba, wb, bb, wc, o):
    x = (jnp.dot(pf[0], w1e[...], preferred_element_type=jnp.float32)
         + b1[...])
    o[...] = _res(x, wa[...], ba[...], wb[...], bb[...], wc[...])


def _tc_init(p4, w1t, b1, wa, ba, wb, bb, wc):
    extras = [w1t, b1, wa, ba, wb, bb, wc]
    return pl.pallas_call(
        _tc_init_body,
        grid=(BS * NP // RB,),
        in_specs=([pl.BlockSpec((1, RB, 4), lambda i: (i // (NP // RB),
                                                       i % (NP // RB), 0))]
                  + [_full_spec(a) for a in extras]),
        out_specs=pl.BlockSpec((RB, MID), lambda i: (i, 0)),
        out_shape=jax.ShapeDtypeStruct((BS * NP, MID), jnp.float32),
        compiler_params=pltpu.CompilerParams(
            dimension_semantics=("parallel",)),
    )(p4, *extras)


def _tc_res_packed_body(pa, pb, at, ab, ba, wb, bb, ct, cb, wf, bf, o):
    xa, xb = pa[...], pb[...]
    ra = jnp.maximum(xa, 0.0)
    rb = jnp.maximum(xb, 0.0)
    dot = lambda x, w: jnp.dot(x, w, preferred_element_type=jnp.float32)
    h = dot(ra, at[...]) + dot(rb, ab[...]) + ba[...]
    h = dot(jnp.maximum(h, 0.0), wb[...]) + bb[...]
    y = dot(xa, ct[...]) + dot(xb, cb[...]) + h
    o[...] = dot(y, wf[...]) + bf[...]


def _tc_transpose_body(x, o):
    o[...] = jnp.transpose(x[...], (0, 2, 1))


def _tc_transpose(tbl):
    return pl.pallas_call(
        _tc_transpose_body,
        grid=(BS, D3 // 512),
        in_specs=[pl.BlockSpec((1, 512, MID), lambda b, j: (b, j, 0))],
        out_specs=pl.BlockSpec((1, MID, 512), lambda b, j: (b, 0, j)),
        out_shape=jax.ShapeDtypeStruct((BS, MID, D3), jnp.float32),
        compiler_params=pltpu.CompilerParams(
            dimension_semantics=("parallel", "parallel")),
    )(tbl)


def _full_spec(arr):
    nd = arr.ndim
    return pl.BlockSpec(arr.shape, lambda i, _nd=nd: (0,) * _nd)


def _tc_res_packed(p1p, pooledp, w):
    rows = p1p.shape[0]
    rb = RB // 2
    in_specs = ([pl.BlockSpec((rb, 128), lambda i: (i, 0))] * 2
                + [_full_spec(a) for a in w])
    return pl.pallas_call(
        _tc_res_packed_body,
        grid=(rows // rb,),
        in_specs=in_specs,
        out_specs=pl.BlockSpec((rb, 128), lambda i: (i, 0)),
        out_shape=jax.ShapeDtypeStruct((rows, 128), jnp.float32),
        input_output_aliases={0: 0},
        compiler_params=pltpu.CompilerParams(
            dimension_semantics=("parallel",)),
    )(p1p, pooledp, *w)


def kernel(p, params):
    p_pad = jnp.pad(p, ((0, 0), (0, NP - NPTS), (0, 0)))
    idx, inv = _sc_counts(p_pad)

    def blk(i):
        return (params['blk%d_fc1_w' % i].T, params['blk%d_fc1_b' % i][None],
                params['blk%d_fc2_w' % i].T, params['blk%d_fc2_b' % i][None],
                params['blk%d_fc3_w' % i].T)

    p4 = jnp.pad(p_pad, ((0, 0), (0, 0), (0, 1)))
    w1t4 = jnp.pad(params['w1'].T, ((0, 1), (0, 0)))
    p1 = _tc_init(p4, w1t4, params['b1'][None], *blk(0))

    eye4 = jnp.eye(4, dtype=jnp.float32)
    bd = lambda w: jnp.kron(eye4, w)
    t4 = lambda b: jnp.tile(b, (1, 4))

    def pack_stage(wa, ba, wb, bb, wc, wf, bf):
        return (bd(wa[:MID]), bd(wa[MID:]), t4(ba), bd(wb), t4(bb),
                bd(wc[:MID]), bd(wc[MID:]), bd(wf), t4(bf))

    eye = jnp.eye(MID, dtype=jnp.float32)
    zb = jnp.zeros((1, MID), jnp.float32)
    zw = jnp.zeros((2 * MID, MID), jnp.float32)
    stages = [pack_stage(*blk(i), eye, zb) for i in range(1, 4)]
    stages.append(pack_stage(*blk(4), params['w2'].T, params['b2'][None]))
    stages.append(pack_stage(zw, zb, jnp.zeros((MID, MID), jnp.float32),
                             zb, zw, eye, zb))
    ws = tuple(jnp.stack([st[j] for st in stages]) for j in range(9))

    def step(carry, w):
        p1_c, _ = carry
        pooled, tbl = _sc_pool(jnp.reshape(p1_c, (BS * NP, MID)), idx, inv)
        pooledp = jnp.reshape(pooled, (BS * NP // 4, 128))
        p1_n = _tc_res_packed(p1_c, pooledp, list(w))
        tblp = jnp.reshape(tbl, (BS, D3 * MID // 128, 128))
        return (p1_n, tblp), None

    p1p = jnp.reshape(p1, (BS * NP // 4, 128))
    (_, tblp), _ = lax.scan(
        step, (p1p, jnp.zeros((BS, D3 * MID // 128, 128), jnp.float32)), ws)
    tbl = jnp.reshape(tblp, (BS, D3, MID))
    return _tc_transpose(tbl).reshape(BS, MID, RESO, RESO, RESO)


# speedup vs baseline: 396.0631x; 1.0384x over previous
import functools

import jax
import jax.numpy as jnp
from jax import lax
from jax.experimental import pallas as pl
from jax.experimental.pallas import tpu as pltpu
from jax.experimental.pallas import tpu_sc as plsc

BS = 4
NPTS = 100000
MID = 32
RESO = 32
D3 = RESO ** 3
NC = 2
NS = 16
CH = 49
TPB = CH * 128
NP = NS * TPB
TROWS = D3 + 16
RPT = D3 // NS
DEN = 1.0 + 0.1 + 0.001
HI = 1.0 - 0.001
RB = 1024


def _mesh():
    return plsc.VectorSubcoreMesh(core_axis_name="c", subcore_axis_name="s")


def _iota16():
    return lax.iota(jnp.int32, 16)


def _fill_rows(ref, nrows, ncols, value):
    iota = _iota16()
    vec = jnp.full((16,), value, jnp.float32)

    def body(r, _):
        rr = jnp.full((16,), r, jnp.int32)
        for cb in range(ncols // 16):
            plsc.store_scatter(ref, [rr, iota + cb * 16], vec)
        return 0

    lax.fori_loop(0, nrows, body, 0)


def _sc_counts_body(p_hbm, idx_out, inv_out, cnt_tbl, pbuf, obuf, ibuf, cbuf,
                    invb):
    c = lax.axis_index("c")
    s = lax.axis_index("s")
    iota = _iota16()
    zeros_i = jnp.zeros((16,), jnp.int32)

    _fill_rows(obuf, 128, 16, 1.0)

    for k in range(NC):
        b = c * NC + k
        base = pl.multiple_of(s * RPT, 8)

        _fill_rows(cbuf, RPT, 16, 0.0)
        pltpu.sync_copy(cbuf, cnt_tbl.at[pl.ds(base, RPT)])

        @pl.when(s == NS - 1)
        def _():
            pltpu.sync_copy(cbuf.at[pl.ds(0, TROWS - D3)],
                            cnt_tbl.at[pl.ds(D3, TROWS - D3)])

        plsc.subcore_barrier()

        def chunk(ch, _):
            off = pl.multiple_of(s * TPB + ch * 128, 8)
            pltpu.sync_copy(p_hbm.at[b, pl.ds(off, 128), :], pbuf)
            for g in range(8):
                rows = jnp.full((16,), g * 16, jnp.int32) + iota

                def coord(col):
                    v = plsc.load_gather(
                        pbuf, [rows, jnp.full((16,), col, jnp.int32)])
                    ncv = jnp.clip(v / DEN + 0.5, 0.0, HI)
                    return (ncv * float(RESO)).astype(jnp.int32)

                qx, qy, qz = coord(0), coord(1), coord(2)
                idxv = qx + RESO * (qy + RESO * qz)
                pos = jnp.full((16,), off + g * 16, jnp.int32) + iota
                idxv = jnp.where(pos < NPTS, idxv, D3)
                ibuf[pl.ds(g * 16, 16)] = idxv
            pltpu.sync_copy(obuf, cnt_tbl.at[ibuf], add=True)
            pltpu.sync_copy(ibuf, idx_out.at[b, s, ch])
            return 0

        lax.fori_loop(0, CH, chunk, 0)
        plsc.subcore_barrier()

        pltpu.sync_copy(cnt_tbl.at[pl.ds(base, RPT)], cbuf)

        def invrow(i, _):
            rows = jnp.full((16,), i * 16, jnp.int32) + iota
            cv = plsc.load_gather(cbuf, [rows, zeros_i])
            invb[pl.ds(i * 16, 16)] = 1.0 / jnp.maximum(cv, 1.0)
            return 0

        lax.fori_loop(0, RPT // 16, invrow, 0)
        pltpu.sync_copy(invb, inv_out.at[b, pl.ds(base, RPT)])
        plsc.subcore_barrier()


def _sc_pool_body(p1_hbm, idx_hbm, inv_hbm, pooled_out, tbl_out, ftbl,
                  rbuf, nbuf, ibuf, invb):
    c = lax.axis_index("c")
    s = lax.axis_index("s")
    iota = _iota16()

    for k in range(NC):
        b = c * NC + k
        base = pl.multiple_of(s * RPT, 8)

        _fill_rows(nbuf, 256, 32, 0.0)
        for zblk in range(RPT // 256):
            pltpu.sync_copy(nbuf, ftbl.at[pl.ds(base + zblk * 256, 256)])

        @pl.when(s == NS - 1)
        def _():
            pltpu.sync_copy(nbuf.at[pl.ds(0, TROWS - D3)],
                            ftbl.at[pl.ds(D3, TROWS - D3)])

        pltpu.sync_copy(idx_hbm.at[b, s], ibuf)
        pltpu.sync_copy(inv_hbm.at[b, pl.ds(base, RPT)], invb)
        plsc.subcore_barrier()

        def scatter(ch, _):
            off = pl.multiple_of(b * NP + s * TPB + ch * 128, 8)
            pltpu.sync_copy(p1_hbm.at[pl.ds(off, 128), :], rbuf)
            pltpu.sync_copy(rbuf, ftbl.at[ibuf.at[ch]], add=True)
            return 0

        lax.fori_loop(0, CH, scatter, 0)
        plsc.subcore_barrier()

        for nb in range(RPT // 256):
            pltpu.sync_copy(ftbl.at[pl.ds(base + nb * 256, 256)], nbuf)

            def norm(i, _, _nb=nb):
                ii = jnp.full((16,), i, jnp.int32)
                spl = plsc.load_gather(invb, [jnp.full((16,), _nb * 256,
                                                       jnp.int32) + ii])
                r0 = plsc.load_gather(nbuf, [ii, iota])
                r1 = plsc.load_gather(nbuf, [ii, iota + 16])
                plsc.store_scatter(nbuf, [ii, iota], r0 * spl)
                plsc.store_scatter(nbuf, [ii, iota + 16], r1 * spl)
                return 0

            lax.fori_loop(0, 256, norm, 0)
            pltpu.sync_copy(nbuf, tbl_out.at[b, pl.ds(base + nb * 256, 256), :])
            pltpu.sync_copy(nbuf, ftbl.at[pl.ds(base + nb * 256, 256)])
        plsc.subcore_barrier()

        def gather(ch, _):
            off = pl.multiple_of(b * NP + s * TPB + ch * 128, 8)
            pltpu.sync_copy(ftbl.at[ibuf.at[ch]], rbuf)
            pltpu.sync_copy(rbuf, pooled_out.at[pl.ds(off, 128), :])
            return 0

        lax.fori_loop(0, CH, gather, 0)
        plsc.subcore_barrier()


def _sc_counts(p_pad):
    return pl.kernel(
        _sc_counts_body,
        out_type=(jax.ShapeDtypeStruct((BS, NS, CH, 128), jnp.int32),
                  jax.ShapeDtypeStruct((BS, D3), jnp.float32)),
        mesh=_mesh(),
        compiler_params=pltpu.CompilerParams(needs_layout_passes=False, use_tc_tiling_on_sc=False),
        scratch_types=[
            pltpu.VMEM_SHARED((TROWS, 16), jnp.float32),
            pltpu.VMEM((128, 4), jnp.float32),
            pltpu.VMEM((128, 16), jnp.float32),
            pltpu.VMEM((128,), jnp.int32),
            pltpu.VMEM((RPT, 16), jnp.float32),
            pltpu.VMEM((RPT,), jnp.float32),
        ],
    )(p_pad)


def _sc_pool(p1, idx, inv):
    return pl.kernel(
        _sc_pool_body,
        out_type=(jax.ShapeDtypeStruct((BS * NP, MID), jnp.float32),
                  jax.ShapeDtypeStruct((BS, D3, MID), jnp.float32)),
        mesh=_mesh(),
        compiler_params=pltpu.CompilerParams(needs_layout_passes=False, use_tc_tiling_on_sc=False),
        scratch_types=[
            pltpu.VMEM_SHARED((TROWS, MID), jnp.float32),
            pltpu.VMEM((128, MID), jnp.float32),
            pltpu.VMEM((256, MID), jnp.float32),
            pltpu.VMEM((CH, 128), jnp.int32),
            pltpu.VMEM((RPT,), jnp.float32),
        ],
    )(p1, idx, inv)


def _res(x, wa, ba, wb, bb, wc):
    h = jnp.maximum(x, 0.0)
    h = jnp.dot(h, wa, preferred_element_type=jnp.float32) + ba
    h = jnp.maximum(h, 0.0)
    h = jnp.dot(h, wb, preferred_element_type=jnp.float32) + bb
    return jnp.dot(x, wc, preferred_element_type=jnp.float32) + h


def _tc_init_body(pf, w1e, b1, wa, ba, wb, bb, wc, o):
    x = (jnp.dot(pf[0], w1e[...], preferred_element_type=jnp.float32)
         + b1[...])
    o[...] = _res(x, wa[...], ba[...], wb[...], bb[...], wc[...])


def _tc_init(p4, w1t, b1, wa, ba, wb, bb, wc):
    extras = [w1t, b1, wa, ba, wb, bb, wc]
    return pl.pallas_call(
        _tc_init_body,
        grid=(BS * NP // RB,),
        in_specs=([pl.BlockSpec((1, RB, 4), lambda i: (i // (NP // RB),
                                                       i % (NP // RB), 0))]
                  + [_full_spec(a) for a in extras]),
        out_specs=pl.BlockSpec((RB, MID), lambda i: (i, 0)),
        out_shape=jax.ShapeDtypeStruct((BS * NP, MID), jnp.float32),
        compiler_params=pltpu.CompilerParams(
            dimension_semantics=("parallel",)),
    )(p4, *extras)


def _tc_res_packed_body(pa, pb, at, ab, ba, wb, bb, ct, cb, wf, bf, o):
    xa, xb = pa[...], pb[...]
    ra = jnp.maximum(xa, 0.0)
    rb = jnp.maximum(xb, 0.0)
    dot = lambda x, w: jnp.dot(x, w, preferred_element_type=jnp.float32)
    h = dot(ra, at[...]) + dot(rb, ab[...]) + ba[...]
    h = dot(jnp.maximum(h, 0.0), wb[...]) + bb[...]
    y = dot(xa, ct[...]) + dot(xb, cb[...]) + h
    o[...] = dot(y, wf[...]) + bf[...]


def _tc_transpose_body(x, o):
    o[...] = jnp.transpose(x[...], (0, 2, 1))


def _tc_transpose(tbl):
    return pl.pallas_call(
        _tc_transpose_body,
        grid=(BS, D3 // 512),
        in_specs=[pl.BlockSpec((1, 512, MID), lambda b, j: (b, j, 0))],
        out_specs=pl.BlockSpec((1, MID, 512), lambda b, j: (b, 0, j)),
        out_shape=jax.ShapeDtypeStruct((BS, MID, D3), jnp.float32),
        compiler_params=pltpu.CompilerParams(
            dimension_semantics=("parallel", "parallel")),
    )(tbl)


def _full_spec(arr):
    nd = arr.ndim
    return pl.BlockSpec(arr.shape, lambda i, _nd=nd: (0,) * _nd)


def _tc_res_packed(p1p, pooledp, w):
    rows = p1p.shape[0]
    rb = RB // 2
    in_specs = ([pl.BlockSpec((rb, 128), lambda i: (i, 0))] * 2
                + [_full_spec(a) for a in w])
    return pl.pallas_call(
        _tc_res_packed_body,
        grid=(rows // rb,),
        in_specs=in_specs,
        out_specs=pl.BlockSpec((rb, 128), lambda i: (i, 0)),
        out_shape=jax.ShapeDtypeStruct((rows, 128), jnp.float32),
        input_output_aliases={0: 0},
        compiler_params=pltpu.CompilerParams(
            dimension_semantics=("parallel",)),
    )(p1p, pooledp, *w)


def kernel(p, params):
    p4 = jnp.pad(p, ((0, 0), (0, NP - NPTS), (0, 1)))
    idx, inv = _sc_counts(p4)

    def blk(i):
        return (params['blk%d_fc1_w' % i].T, params['blk%d_fc1_b' % i][None],
                params['blk%d_fc2_w' % i].T, params['blk%d_fc2_b' % i][None],
                params['blk%d_fc3_w' % i].T)

    w1t4 = jnp.pad(params['w1'].T, ((0, 1), (0, 0)))
    p1 = _tc_init(p4, w1t4, params['b1'][None], *blk(0))

    eye4 = jnp.eye(4, dtype=jnp.float32)
    bd = lambda w: jnp.kron(eye4, w)
    t4 = lambda b: jnp.tile(b, (1, 4))

    def pack_stage(wa, ba, wb, bb, wc, wf, bf):
        return (bd(wa[:MID]), bd(wa[MID:]), t4(ba), bd(wb), t4(bb),
                bd(wc[:MID]), bd(wc[MID:]), bd(wf), t4(bf))

    eye = jnp.eye(MID, dtype=jnp.float32)
    zb = jnp.zeros((1, MID), jnp.float32)
    zw = jnp.zeros((2 * MID, MID), jnp.float32)
    stages = [pack_stage(*blk(i), eye, zb) for i in range(1, 4)]
    stages.append(pack_stage(*blk(4), params['w2'].T, params['b2'][None]))
    stages.append(pack_stage(zw, zb, jnp.zeros((MID, MID), jnp.float32),
                             zb, zw, eye, zb))
    ws = tuple(jnp.stack([st[j] for st in stages]) for j in range(9))

    def step(carry, w):
        p1_c, _ = carry
        pooled, tbl = _sc_pool(jnp.reshape(p1_c, (BS * NP, MID)), idx, inv)
        pooledp = jnp.reshape(pooled, (BS * NP // 4, 128))
        p1_n = _tc_res_packed(p1_c, pooledp, list(w))
        tblp = jnp.reshape(tbl, (BS, D3 * MID // 128, 128))
        return (p1_n, tblp), None

    p1p = jnp.reshape(p1, (BS * NP // 4, 128))
    (_, tblp), _ = lax.scan(
        step, (p1p, jnp.zeros((BS, D3 * MID // 128, 128), jnp.float32)), ws)
    tbl = jnp.reshape(tblp, (BS, D3, MID))
    return _tc_transpose(tbl).reshape(BS, MID, RESO, RESO, RESO)
